# baseline (speedup 1.0000x reference)
import functools

import jax
import jax.numpy as jnp
from jax import lax
from jax.experimental import pallas as pl
from jax.experimental.pallas import tpu as pltpu
from jax.experimental.pallas import tpu_sc as plsc


def kernel(center_ids, context_ids, center_weight, center_biase, context_weight, context_biase):
    del context_ids
    B = center_ids.shape[0]
    D = center_weight.shape[1]
    L = 16

    info = plsc.get_sparse_core_info()
    NC, NS = info.num_cores, info.num_subcores
    NW = NC * NS
    n = B // NW

    ids = center_ids.astype(jnp.int32)
    cb_flat = center_biase.reshape(-1)
    xb_flat = context_biase.reshape(-1)
    mesh = plsc.VectorSubcoreMesh(core_axis_name="c", subcore_axis_name="s")

    @functools.partial(
        pl.kernel,
        mesh=mesh,
        compiler_params=pltpu.CompilerParams(
            needs_layout_passes=False,
            use_tc_tiling_on_sc=False,
        ),
        out_type=jax.ShapeDtypeStruct((B,), jnp.float32),
        scratch_types=[
            pltpu.VMEM((n,), jnp.int32),
            pltpu.VMEM((n, D), jnp.float32),
            pltpu.VMEM((n, D), jnp.float32),
            pltpu.VMEM((n,), jnp.float32),
            pltpu.VMEM((n,), jnp.float32),
            pltpu.VMEM((n,), jnp.float32),
            pltpu.SemaphoreType.DMA,
        ],
    )
    def glove_sc(ids_hbm, cw_hbm, cb_hbm, xw_hbm, xb_hbm, out_hbm,
                 idx_v, cw_v, xw_v, cb_v, xb_v, out_v, sem):
        wid = lax.axis_index("s") * NC + lax.axis_index("c")
        base = wid * n

        pltpu.sync_copy(ids_hbm.at[pl.ds(base, n)], idx_v)
        g1 = pltpu.async_copy(cw_hbm.at[idx_v], cw_v, sem)
        g2 = pltpu.async_copy(xw_hbm.at[idx_v], xw_v, sem)
        g3 = pltpu.async_copy(cb_hbm.at[idx_v], cb_v, sem)
        g4 = pltpu.async_copy(xb_hbm.at[idx_v], xb_v, sem)
        g1.wait()
        g2.wait()
        g3.wait()
        g4.wait()

        lanes = lax.iota(jnp.int32, L)

        def body(t, _):
            o = t * L
            acc = cb_v[pl.ds(o, L)] + xb_v[pl.ds(o, L)]
            for k in range(L):
                p = cw_v[o + k, pl.ds(0, L)] * xw_v[o + k, pl.ds(0, L)]
                p = p + cw_v[o + k, pl.ds(L, L)] * xw_v[o + k, pl.ds(L, L)]
                s = jnp.sum(p)
                acc = acc + jnp.where(lanes == k, s, jnp.float32(0.0))
            out_v[pl.ds(o, L)] = acc
            return 0

        lax.fori_loop(0, n // L, body, 0)
        pltpu.sync_copy(out_v, out_hbm.at[pl.ds(base, n)])

    out = glove_sc(ids, center_weight, cb_flat, context_weight, xb_flat)
    return out.reshape(B, 1)
